# baseline (speedup 1.0000x reference)
import functools

import jax
import jax.numpy as jnp
from jax import lax
from jax.experimental import pallas as pl
from jax.experimental.pallas import tpu as pltpu
from jax.experimental.pallas import tpu_sc as plsc

_NC = 2
_NS = 16
_NW = _NC * _NS
_CHUNK = 128
_PD = 128
_BLK = 128


def _diag_vecs():
    iota16 = lax.iota(jnp.int32, 16)
    half_iota = lax.shift_right_logical(iota16, 1)
    h64 = lax.shift_left(iota16 & 1, 6)
    return iota16, half_iota, h64


@functools.cache
def _build_repack(V, D):
    nblk_full = V // _BLK
    vtail = V - nblk_full * _BLK
    per_w = nblk_full // _NW
    rem = nblk_full - per_w * _NW
    assert per_w % 4 == 0 and rem + (1 if vtail else 0) <= _NW
    mesh = plsc.VectorSubcoreMesh(core_axis_name="c", subcore_axis_name="s")

    @functools.partial(
        pl.kernel,
        mesh=mesh,
        out_type=jax.ShapeDtypeStruct((V // 2, _PD), jnp.float32),
        scratch_types=[
            pltpu.VMEM((4, D, _BLK), jnp.float32),
            pltpu.VMEM((2, _BLK // 2, _PD), jnp.float32),
            pltpu.SemaphoreType.DMA,
            pltpu.SemaphoreType.DMA,
            pltpu.SemaphoreType.DMA,
            pltpu.SemaphoreType.DMA,
            pltpu.SemaphoreType.DMA,
            pltpu.SemaphoreType.DMA,
        ],
        compiler_params=pltpu.CompilerParams(
            use_tc_tiling_on_sc=True, needs_layout_passes=False,
            disable_bounds_checks=True),
    )
    def k(embt_hbm, pairs_hbm, slab, pblk,
          rs0, rs1, rs2, rs3, ws0, ws1):
        wid = lax.axis_index("s") * _NC + lax.axis_index("c")
        base = wid * per_w
        iota16, half_iota, h64 = _diag_vecs()
        rsems = (rs0, rs1, rs2, rs3)
        wsems = (ws0, ws1)
        f0s = tuple(range(0, D, 16))

        def transpose(sl, pp, nvt):
            src = slab.at[sl]
            dst = pblk.at[pp]

            @plsc.parallel_loop(0, nvt)
            def _(vt):
                svec = iota16 + 16 * vt
                pvec = half_iota + 8 * vt
                for kk in range(16):
                    ck = (iota16 + kk) & 15
                    prek = h64 + ck
                    vals = [plsc.load_gather(src, [ck + f0, svec])
                            for f0 in f0s]
                    for f0, v in zip(f0s, vals):
                        plsc.store_scatter(dst, [pvec, prek + f0], v)

        def fire_read(sl, b):
            pltpu.async_copy(
                embt_hbm.at[:, pl.ds(b * _BLK, _BLK)], slab.at[sl],
                rsems[sl])

        def wait_read(sl, b):
            pltpu.make_async_copy(
                embt_hbm.at[:, pl.ds(b * _BLK, _BLK)], slab.at[sl],
                rsems[sl]).wait()

        def fire_write(pp, b):
            pltpu.async_copy(
                pblk.at[pp], pairs_hbm.at[pl.ds(b * (_BLK // 2), _BLK // 2)],
                wsems[pp])

        def wait_write(pp, b):
            pltpu.make_async_copy(
                pblk.at[pp], pairs_hbm.at[pl.ds(b * (_BLK // 2), _BLK // 2)],
                wsems[pp]).wait()

        nvt_full = _BLK // 16

        for j in range(3):
            fire_read(j, base + j)

        def body(i, carry):
            for j in range(4):
                t = 4 * i + j
                b = base + t

                @pl.when(t + 3 <= per_w - 1)
                def _():
                    fire_read((j + 3) % 4, b + 3)

                @pl.when(t >= 2)
                def _():
                    wait_write(j % 2, b - 2)

                wait_read(j, b)
                transpose(j, j % 2, nvt_full)
                fire_write(j % 2, b)
            return carry

        lax.fori_loop(0, per_w // 4, body, 0)
        wait_write(0, base + per_w - 2)
        wait_write(1, base + per_w - 1)

        @pl.when(wid < rem)
        def _():
            b = nblk_full - rem + wid
            fire_read(0, b)
            wait_read(0, b)
            transpose(0, 0, nvt_full)
            fire_write(0, b)
            wait_write(0, b)

        if vtail:
            @pl.when(wid == rem)
            def _():
                b = lax.convert_element_type(nblk_full, jnp.int32)
                fire_read(0, b)
                wait_read(0, b)
                transpose(0, 0, vtail // 16)
                pltpu.sync_copy(
                    pblk.at[0, pl.ds(0, vtail // 2)],
                    pairs_hbm.at[pl.ds(nblk_full * (_BLK // 2), vtail // 2)])

    return k


@functools.cache
def _build_gather(V, D, T, S):
    assert S == _NW * _CHUNK and T % 4 == 0
    mesh = plsc.VectorSubcoreMesh(core_axis_name="c", subcore_axis_name="s")

    @functools.partial(
        pl.kernel,
        mesh=mesh,
        out_type=jax.ShapeDtypeStruct((T, D, S), jnp.float32),
        scratch_types=[
            pltpu.VMEM((T, _CHUNK), jnp.int32),
            pltpu.VMEM((4, _CHUNK), jnp.int32),
            pltpu.VMEM((4, _CHUNK, _PD), jnp.float32),
            pltpu.VMEM((2, D, _CHUNK), jnp.float32),
            pltpu.SemaphoreType.DMA,
            pltpu.SemaphoreType.DMA,
            pltpu.SemaphoreType.DMA,
            pltpu.SemaphoreType.DMA,
            pltpu.SemaphoreType.DMA,
            pltpu.SemaphoreType.DMA,
        ],
        compiler_params=pltpu.CompilerParams(
            use_tc_tiling_on_sc=True, needs_layout_passes=False),
    )
    def k(xt_hbm, pairs_hbm, out_hbm, idx_v, pring, grows, tblk,
          gs0, gs1, gs2, gs3, os0, os1):
        wid = lax.axis_index("s") * _NC + lax.axis_index("c")
        s0 = wid * _CHUNK
        iota16, _, _ = _diag_vecs()
        gsems = (gs0, gs1, gs2, gs3)
        osems = (os0, os1)
        f0s = tuple(range(0, D, 16))
        pltpu.sync_copy(xt_hbm.at[:, pl.ds(s0, _CHUNK)], idx_v)

        def pidx_compute(sl, t):
            for jj in range(_CHUNK // 16):
                v = idx_v[t, pl.ds(16 * jj, 16)]
                pring[sl, pl.ds(16 * jj, 16)] = lax.shift_right_logical(v, 1)

        def fire_gather(sl, t):
            pltpu.async_copy(
                pairs_hbm.at[pring.at[sl]], grows.at[sl], gsems[sl])

        def wait_gather(sl, t):
            pltpu.make_async_copy(
                pairs_hbm.at[pring.at[sl]], grows.at[sl], gsems[sl]).wait()

        def fire_out(pp, t):
            pltpu.async_copy(
                tblk.at[pp], out_hbm.at[t, :, pl.ds(s0, _CHUNK)], osems[pp])

        def wait_out(pp, t):
            pltpu.make_async_copy(
                tblk.at[pp], out_hbm.at[t, :, pl.ds(s0, _CHUNK)],
                osems[pp]).wait()

        def transpose(sl, pp, t):
            src = grows.at[sl]
            dst = tblk.at[pp]

            @plsc.parallel_loop(0, _CHUNK // 16)
            def _(st):
                svec = iota16 + 16 * st
                hraw = idx_v[t, pl.ds(16 * st, 16)]
                hv = lax.shift_left(hraw & 1, 6)
                for kk in range(16):
                    ck = (iota16 + kk) & 15
                    colb = hv + ck
                    vals = [plsc.load_gather(src, [svec, colb + f0])
                            for f0 in f0s]
                    for f0, v in zip(f0s, vals):
                        plsc.store_scatter(dst, [ck + f0, svec], v)

        for j in range(3):
            pidx_compute(j, j)
            fire_gather(j, j)

        def body(i, carry):
            for j in range(4):
                t = 4 * i + j

                @pl.when(t + 3 <= T - 1)
                def _():
                    pidx_compute((j + 3) % 4, t + 3)
                    fire_gather((j + 3) % 4, t + 3)

                @pl.when(t >= 2)
                def _():
                    wait_out(j % 2, t - 2)

                wait_gather(j, t)
                transpose(j, j % 2, t)
                fire_out(j % 2, t)
            return carry

        lax.fori_loop(0, T // 4, body, 0)
        wait_out(0, T - 2)
        wait_out(1, T - 1)

    return k


def kernel(x, embedding):
    S, T = x.shape
    V, D = embedding.shape
    xt = x.T
    embt = embedding.T
    pairs = _build_repack(V, D)(embt)
    out_t = _build_gather(V, D, T, S)(xt, pairs)
    return jnp.transpose(out_t, (2, 0, 1))


# speedup vs baseline: 1.2013x; 1.2013x over previous
import functools

import jax
import jax.numpy as jnp
from jax import lax
from jax.experimental import pallas as pl
from jax.experimental.pallas import tpu as pltpu
from jax.experimental.pallas import tpu_sc as plsc

_NC = 2
_NS = 16
_NW = _NC * _NS
_CHUNK = 128
_PD = 128
_BLK = 256


def _diag_vecs():
    iota16 = lax.iota(jnp.int32, 16)
    half_iota = lax.shift_right_logical(iota16, 1)
    h64 = lax.shift_left(iota16 & 1, 6)
    return iota16, half_iota, h64


@functools.cache
def _build_repack(V, D):
    nblk_full = V // _BLK
    vtail = V - nblk_full * _BLK
    per_w = nblk_full // _NW
    rem = nblk_full - per_w * _NW
    assert per_w % 2 == 0 and rem + (1 if vtail else 0) <= _NW
    mesh = plsc.VectorSubcoreMesh(core_axis_name="c", subcore_axis_name="s")

    @functools.partial(
        pl.kernel,
        mesh=mesh,
        out_type=jax.ShapeDtypeStruct((V // 2, _PD), jnp.float32),
        scratch_types=[
            pltpu.VMEM((2, D, _BLK), jnp.float32),
            pltpu.VMEM((2, _BLK // 2, _PD), jnp.float32),
            pltpu.SemaphoreType.DMA,
            pltpu.SemaphoreType.DMA,
            pltpu.SemaphoreType.DMA,
        ],
        compiler_params=pltpu.CompilerParams(
            use_tc_tiling_on_sc=True, needs_layout_passes=False,
            disable_bounds_checks=True),
    )
    def k(embt_hbm, pairs_hbm, slab, pblk, rsem0, rsem1, wsem):
        wid = lax.axis_index("s") * _NC + lax.axis_index("c")
        base = wid * per_w
        iota16, half_iota, h64 = _diag_vecs()
        rsems = (rsem0, rsem1)

        def transpose(p, nvt):
            src = slab.at[p]
            dst = pblk.at[p]
            f0s = tuple(range(0, D, 16))

            @plsc.parallel_loop(0, nvt)
            def _(vt):
                svec = iota16 + 16 * vt
                pvec = half_iota + 8 * vt
                for kk in range(16):
                    ck = (iota16 + kk) & 15
                    prek = h64 + ck
                    vals = [plsc.load_gather(src, [ck + f0, svec])
                            for f0 in f0s]
                    for f0, v in zip(f0s, vals):
                        plsc.store_scatter(dst, [pvec, prek + f0], v)

        def fire_read(p, b):
            pltpu.async_copy(
                embt_hbm.at[:, pl.ds(b * _BLK, _BLK)], slab.at[p], rsems[p])

        def wait_read(p, b):
            pltpu.make_async_copy(
                embt_hbm.at[:, pl.ds(b * _BLK, _BLK)], slab.at[p],
                rsems[p]).wait()

        def fire_write(p, b):
            pltpu.async_copy(
                pblk.at[p], pairs_hbm.at[pl.ds(b * (_BLK // 2), _BLK // 2)],
                wsem)

        def wait_write(p, b):
            pltpu.make_async_copy(
                pblk.at[p], pairs_hbm.at[pl.ds(b * (_BLK // 2), _BLK // 2)],
                wsem).wait()

        nvt_full = _BLK // 16

        fire_read(0, base)
        fire_read(1, base + 1)
        wait_read(0, base)
        transpose(0, nvt_full)
        fire_write(0, base)

        def body(i, carry):
            for par in (1, 0):
                b = base + 2 * i + (1 if par == 1 else 2)
                fire_read(1 - par, b + 1)
                wait_read(par, b)
                transpose(par, nvt_full)
                fire_write(par, b)
                wait_write(1 - par, b - 1)
            return carry

        lax.fori_loop(0, (per_w - 2) // 2, body, 0)

        bl = base + per_w - 1
        wait_read(1, bl)
        transpose(1, nvt_full)
        fire_write(1, bl)
        wait_write(0, bl - 1)
        wait_write(1, bl)

        @pl.when(wid < rem)
        def _():
            b = nblk_full - rem + wid
            fire_read(0, b)
            wait_read(0, b)
            transpose(0, nvt_full)
            fire_write(0, b)
            wait_write(0, b)

        if vtail:
            @pl.when(wid == rem)
            def _():
                b = lax.convert_element_type(nblk_full, jnp.int32)
                tsrc = embt_hbm.at[:, pl.ds(b * _BLK, 128)]
                tdst = slab.at[0, :, pl.ds(0, 128)]
                pltpu.async_copy(tsrc, tdst, rsems[0])
                pltpu.make_async_copy(tsrc, tdst, rsems[0]).wait()
                transpose(0, vtail // 16)
                pltpu.sync_copy(
                    pblk.at[0, pl.ds(0, vtail // 2)],
                    pairs_hbm.at[pl.ds(nblk_full * (_BLK // 2), vtail // 2)])

    return k


@functools.cache
def _build_gather(V, D, T, S):
    assert S == _NW * _CHUNK and T % 2 == 0
    mesh = plsc.VectorSubcoreMesh(core_axis_name="c", subcore_axis_name="s")

    @functools.partial(
        pl.kernel,
        mesh=mesh,
        out_type=jax.ShapeDtypeStruct((T, D, S), jnp.float32),
        scratch_types=[
            pltpu.VMEM((T, _CHUNK), jnp.int32),
            pltpu.VMEM((T, _CHUNK), jnp.int32),
            pltpu.VMEM((2, _CHUNK, _PD), jnp.float32),
            pltpu.VMEM((2, D, _CHUNK), jnp.float32),
            pltpu.SemaphoreType.DMA,
            pltpu.SemaphoreType.DMA,
            pltpu.SemaphoreType.DMA,
        ],
        compiler_params=pltpu.CompilerParams(
            use_tc_tiling_on_sc=True, needs_layout_passes=False),
    )
    def k(xt_hbm, pairs_hbm, out_hbm, idx_v, pidx, grows, tblk,
          gsem0, gsem1, osem):
        wid = lax.axis_index("s") * _NC + lax.axis_index("c")
        s0 = wid * _CHUNK
        iota16, _, _ = _diag_vecs()
        pltpu.sync_copy(xt_hbm.at[:, pl.ds(s0, _CHUNK)], idx_v)

        def pidx_body(t, carry):
            for j in range(_CHUNK // 16):
                v = idx_v[t, pl.ds(16 * j, 16)]
                pidx[t, pl.ds(16 * j, 16)] = lax.shift_right_logical(v, 1)
            return carry

        lax.fori_loop(0, T, pidx_body, 0)

        gsems = (gsem0, gsem1)

        def fire_gather(p, t):
            pltpu.async_copy(pairs_hbm.at[pidx.at[t]], grows.at[p], gsems[p])

        def wait_gather(p, t):
            pltpu.make_async_copy(
                pairs_hbm.at[pidx.at[t]], grows.at[p], gsems[p]).wait()

        def fire_out(p, t):
            pltpu.async_copy(
                tblk.at[p], out_hbm.at[t, :, pl.ds(s0, _CHUNK)], osem)

        def wait_out(p, t):
            pltpu.make_async_copy(
                tblk.at[p], out_hbm.at[t, :, pl.ds(s0, _CHUNK)], osem).wait()

        def transpose(p, t):
            src = grows.at[p]
            dst = tblk.at[p]
            f0s = tuple(range(0, D, 16))

            @plsc.parallel_loop(0, _CHUNK // 16)
            def _(st):
                svec = iota16 + 16 * st
                hraw = idx_v[t, pl.ds(16 * st, 16)]
                hv = lax.shift_left(hraw & 1, 6)
                for kk in range(16):
                    ck = (iota16 + kk) & 15
                    colb = hv + ck
                    vals = [plsc.load_gather(src, [svec, colb + f0])
                            for f0 in f0s]
                    for f0, v in zip(f0s, vals):
                        plsc.store_scatter(dst, [ck + f0, svec], v)

        fire_gather(0, 0)
        fire_gather(1, 1)
        wait_gather(0, 0)
        transpose(0, 0)
        fire_out(0, 0)

        def body(i, carry):
            for par, off in ((1, 1), (0, 2)):
                tc = 2 * i + off
                fire_gather(1 - par, tc + 1)
                wait_gather(par, tc)
                transpose(par, tc)
                fire_out(par, tc)
                wait_out(1 - par, tc - 1)
            return carry

        lax.fori_loop(0, (T - 2) // 2, body, 0)

        tl = T - 1
        wait_gather(1, tl)
        transpose(1, tl)
        fire_out(1, tl)
        wait_out(0, tl - 1)
        wait_out(1, tl)

    return k


def kernel(x, embedding):
    S, T = x.shape
    V, D = embedding.shape
    xt = x.T
    embt = embedding.T
    pairs = _build_repack(V, D)(embt)
    out_t = _build_gather(V, D, T, S)(xt, pairs)
    return jnp.transpose(out_t, (2, 0, 1))
